# baseline (speedup 1.0000x reference)
import functools

import jax
import jax.numpy as jnp
from jax import lax
from jax.experimental import pallas as pl
from jax.experimental.pallas import tpu as pltpu
from jax.experimental.pallas import tpu_sc as plsc

NUM_EXPERTS = 8
TOP_K = 2
T_BLOCK = 256
SC_CHUNK = 16
NC, NS = 2, 16
NW = NC * NS


def _sc_row_gather(table, idx, n_rows):
    d = table.shape[1]
    dtype = table.dtype
    rows_per_w = n_rows // NW
    n_chunks = rows_per_w // SC_CHUNK
    mesh = plsc.VectorSubcoreMesh(core_axis_name="c", subcore_axis_name="s")

    @functools.partial(
        pl.kernel,
        out_type=jax.ShapeDtypeStruct((n_rows, d), dtype),
        mesh=mesh,
        scratch_types=[
            pltpu.VMEM((SC_CHUNK,), jnp.int32),
            pltpu.VMEM((SC_CHUNK,), jnp.int32),
            pltpu.VMEM((SC_CHUNK, d), dtype),
            pltpu.VMEM((SC_CHUNK, d), dtype),
            pltpu.SemaphoreType.DMA,
            pltpu.SemaphoreType.DMA,
            pltpu.SemaphoreType.DMA,
            pltpu.SemaphoreType.DMA,
        ],
    )
    def gather_kernel(table_hbm, idx_hbm, out_hbm, idx_v0, idx_v1, rows_v0,
                      rows_v1, g_sem0, g_sem1, w_sem0, w_sem1):
        wid = lax.axis_index("s") * NC + lax.axis_index("c")

        @pl.loop(0, n_chunks, step=2)
        def _(c):
            base0 = wid * rows_per_w + c * SC_CHUNK
            base1 = base0 + SC_CHUNK
            pltpu.sync_copy(idx_hbm.at[pl.ds(base0, SC_CHUNK)], idx_v0)
            g0 = pltpu.async_copy(table_hbm.at[idx_v0], rows_v0, g_sem0)
            pltpu.sync_copy(idx_hbm.at[pl.ds(base1, SC_CHUNK)], idx_v1)
            g1 = pltpu.async_copy(table_hbm.at[idx_v1], rows_v1, g_sem1)
            g0.wait()
            w0 = pltpu.async_copy(rows_v0, out_hbm.at[pl.ds(base0, SC_CHUNK)],
                                  w_sem0)
            g1.wait()
            w1 = pltpu.async_copy(rows_v1, out_hbm.at[pl.ds(base1, SC_CHUNK)],
                                  w_sem1)
            w0.wait()
            w1.wait()

    return gather_kernel(table, idx)


def _sc_dual_row_gather(table, idx_a, idx_b):
    d = table.shape[1]
    dtype = table.dtype
    n_rows = idx_a.shape[0]
    rows_per_w = n_rows // NW
    n_chunks = rows_per_w // SC_CHUNK
    mesh = plsc.VectorSubcoreMesh(core_axis_name="c", subcore_axis_name="s")

    @functools.partial(
        pl.kernel,
        out_type=(jax.ShapeDtypeStruct((n_rows, d), dtype),
                  jax.ShapeDtypeStruct((n_rows, d), dtype)),
        mesh=mesh,
        scratch_types=[
            pltpu.VMEM((SC_CHUNK,), jnp.int32),
            pltpu.VMEM((SC_CHUNK,), jnp.int32),
            pltpu.VMEM((SC_CHUNK, d), dtype),
            pltpu.VMEM((SC_CHUNK, d), dtype),
            pltpu.SemaphoreType.DMA,
            pltpu.SemaphoreType.DMA,
            pltpu.SemaphoreType.DMA,
            pltpu.SemaphoreType.DMA,
        ],
    )
    def dual_gather_kernel(table_hbm, ia_hbm, ib_hbm, outa_hbm, outb_hbm,
                           ia_v, ib_v, rows_a, rows_b, ga_sem, gb_sem,
                           wa_sem, wb_sem):
        wid = lax.axis_index("s") * NC + lax.axis_index("c")

        @pl.loop(0, n_chunks)
        def _(c):
            base = wid * rows_per_w + c * SC_CHUNK
            pltpu.sync_copy(ia_hbm.at[pl.ds(base, SC_CHUNK)], ia_v)
            ga = pltpu.async_copy(table_hbm.at[ia_v], rows_a, ga_sem)
            pltpu.sync_copy(ib_hbm.at[pl.ds(base, SC_CHUNK)], ib_v)
            gb = pltpu.async_copy(table_hbm.at[ib_v], rows_b, gb_sem)
            ga.wait()
            wa = pltpu.async_copy(rows_a, outa_hbm.at[pl.ds(base, SC_CHUNK)],
                                  wa_sem)
            gb.wait()
            wb = pltpu.async_copy(rows_b, outb_hbm.at[pl.ds(base, SC_CHUNK)],
                                  wb_sem)
            wa.wait()
            wb.wait()

    return dual_gather_kernel(table, idx_a, idx_b)


def _sc_dispatch_scatter(hs, pos_a, pos_b, capacity):
    n_tok, d = hs.shape
    rows_per_w = n_tok // NW
    n_chunks = rows_per_w // SC_CHUNK
    mesh = plsc.VectorSubcoreMesh(core_axis_name="c", subcore_axis_name="s")

    @functools.partial(
        pl.kernel,
        out_type=jax.ShapeDtypeStruct((capacity, d), hs.dtype),
        mesh=mesh,
        scratch_types=[
            pltpu.VMEM((SC_CHUNK,), jnp.int32),
            pltpu.VMEM((SC_CHUNK,), jnp.int32),
            pltpu.VMEM((SC_CHUNK,), jnp.int32),
            pltpu.VMEM((SC_CHUNK,), jnp.int32),
            pltpu.VMEM((SC_CHUNK, d), hs.dtype),
            pltpu.VMEM((SC_CHUNK, d), hs.dtype),
            pltpu.SemaphoreType.DMA,
            pltpu.SemaphoreType.DMA,
            pltpu.SemaphoreType.DMA,
            pltpu.SemaphoreType.DMA,
            pltpu.SemaphoreType.DMA,
            pltpu.SemaphoreType.DMA,
        ],
    )
    def scatter_kernel(hs_hbm, pa_hbm, pb_hbm, out_hbm, ia0, ib0, ia1, ib1,
                       rows0, rows1, r_sem0, r_sem1, a_sem0, a_sem1, b_sem0,
                       b_sem1):
        wid = lax.axis_index("s") * NC + lax.axis_index("c")
        idx_a = (ia0, ia1)
        idx_b = (ib0, ib1)
        rows = (rows0, rows1)
        r_sem = (r_sem0, r_sem1)
        a_sem = (a_sem0, a_sem1)
        b_sem = (b_sem0, b_sem1)

        @pl.loop(0, n_chunks, step=2)
        def _(c):
            scats = []
            for k in range(2):
                base = wid * rows_per_w + c * SC_CHUNK + k * SC_CHUNK
                r = pltpu.async_copy(hs_hbm.at[pl.ds(base, SC_CHUNK)],
                                     rows[k], r_sem[k])
                pltpu.sync_copy(pa_hbm.at[pl.ds(base, SC_CHUNK)], idx_a[k])
                pltpu.sync_copy(pb_hbm.at[pl.ds(base, SC_CHUNK)], idx_b[k])
                r.wait()
                scats.append(pltpu.async_copy(rows[k], out_hbm.at[idx_a[k]],
                                              a_sem[k]))
                scats.append(pltpu.async_copy(rows[k], out_hbm.at[idx_b[k]],
                                              b_sem[k]))
            for cp in scats:
                cp.wait()

    return scatter_kernel(hs, pos_a, pos_b)


def _grouped_mlp_body(meta_ref, x_ref, wgu_ref, wd_ref, out_ref):
    b = pl.program_id(0)
    e = meta_ref[b]

    @pl.when(e < NUM_EXPERTS)
    def _compute():
        inter = wd_ref.shape[1]
        x = x_ref[...].astype(jnp.bfloat16)
        gu = jnp.dot(x, wgu_ref[0].astype(jnp.bfloat16),
                     preferred_element_type=jnp.float32)
        gate = gu[:, :inter]
        up = gu[:, inter:]
        h = (gate * jax.nn.sigmoid(gate)) * up
        out = jnp.dot(h.astype(jnp.bfloat16), wd_ref[0].astype(jnp.bfloat16),
                      preferred_element_type=jnp.float32)
        half = out.shape[1] // 2
        lo = jax.lax.bitcast_convert_type(
            out[:, :half].astype(jnp.bfloat16), jnp.uint16).astype(jnp.uint32)
        hi = jax.lax.bitcast_convert_type(
            out[:, half:].astype(jnp.bfloat16), jnp.uint16).astype(jnp.uint32)
        out_ref[...] = jax.lax.bitcast_convert_type(lo | (hi << 16),
                                                    jnp.int32)


def _unpack_bf16_pair(p):
    u = jax.lax.bitcast_convert_type(p, jnp.uint32)
    lo = jax.lax.bitcast_convert_type((u & 0xFFFF).astype(jnp.uint16),
                                      jnp.bfloat16)
    hi = jax.lax.bitcast_convert_type((u >> 16).astype(jnp.uint16),
                                      jnp.bfloat16)
    return lo.astype(jnp.float32), hi.astype(jnp.float32)


def _pair_add_body(a0_ref, b0_ref, a1_ref, b1_ref, w_ref, out_ref):
    h = pl.program_id(0)
    w = w_ref[...]
    half = out_ref.shape[1] // 2

    def _emit(a_ref, b_ref):
        a_lo, a_hi = _unpack_bf16_pair(a_ref[...])
        b_lo, b_hi = _unpack_bf16_pair(b_ref[...])
        out_ref[:, :half] = a_lo * w[:, 0:1] + b_lo * w[:, 1:2]
        out_ref[:, half:] = a_hi * w[:, 0:1] + b_hi * w[:, 1:2]

    @pl.when(h == 0)
    def _h0():
        _emit(a0_ref, b0_ref)

    @pl.when(h == 1)
    def _h1():
        _emit(a1_ref, b1_ref)


N_PIPE = 2


def kernel(hidden_states, routing_weights, router_indices, gate_up_proj,
           down_proj):
    b, s, hidden = hidden_states.shape
    n_tok = b * s
    num_experts, _, two_inter = gate_up_proj.shape
    inter = two_inter // 2
    n_h = n_tok // N_PIPE
    a_h = n_h * TOP_K
    cap_h = a_h + num_experts * T_BLOCK
    nb_h = cap_h // T_BLOCK

    hs = hidden_states.reshape(n_tok, hidden)
    rw3 = routing_weights.reshape(N_PIPE, n_h, num_experts)
    ri3 = router_indices.reshape(N_PIPE, n_h, TOP_K).astype(jnp.int32)

    eids = jnp.arange(num_experts, dtype=jnp.int32)
    sel = ri3[..., None] == eids
    w_pair = jnp.sum(jnp.where(sel, rw3[:, :, None, :], 0.0), axis=3)
    dup = ri3[:, :, 1] == ri3[:, :, 0]
    w_pair = jnp.concatenate(
        [w_pair[:, :, 0:1],
         jnp.where(dup[:, :, None], 0.0, w_pair[:, :, 1:2])], axis=2)

    onehot = sel.reshape(N_PIPE, a_h, num_experts)
    csum = jnp.cumsum(onehot.astype(jnp.int32), axis=1)
    counts = csum[:, -1]
    rank_a = jnp.sum(jnp.where(onehot, csum, 0), axis=2) - 1
    padded_counts = ((counts + T_BLOCK - 1) // T_BLOCK) * T_BLOCK
    padded_starts = jnp.concatenate(
        [jnp.zeros((N_PIPE, 1), jnp.int32),
         jnp.cumsum(padded_counts, axis=1)[:, :-1].astype(jnp.int32)], axis=1)
    start_a = jnp.sum(jnp.where(onehot, padded_starts[:, None, :], 0), axis=2)
    pos = (start_a + rank_a).astype(jnp.int32)
    padded_total = jnp.sum(padded_counts, axis=1)
    blk_rows = jnp.arange(nb_h, dtype=jnp.int32) * T_BLOCK
    be = jnp.sum((padded_starts[:, None, :] <=
                  blk_rows[None, :, None]).astype(jnp.int32), axis=2) - 1
    block_meta = jnp.where(blk_rows[None, :] < padded_total[:, None], be,
                           num_experts).astype(jnp.int32)

    grid_spec = pltpu.PrefetchScalarGridSpec(
        num_scalar_prefetch=1,
        grid=(nb_h,),
        in_specs=[
            pl.BlockSpec((T_BLOCK, hidden), lambda i, m: (i, 0)),
            pl.BlockSpec((1, hidden, two_inter),
                         lambda i, m: (jnp.minimum(m[i], NUM_EXPERTS - 1), 0, 0)),
            pl.BlockSpec((1, inter, hidden),
                         lambda i, m: (jnp.minimum(m[i], NUM_EXPERTS - 1), 0, 0)),
        ],
        out_specs=pl.BlockSpec((T_BLOCK, hidden // 2), lambda i, m: (i, 0)),
    )

    gathered = []
    for h in range(N_PIPE):
        pos2 = pos[h].reshape(n_h, TOP_K)
        x_sorted = _sc_dispatch_scatter(hs[h * n_h:(h + 1) * n_h],
                                        pos2[:, 0], pos2[:, 1], cap_h)
        out_sorted = pl.pallas_call(
            _grouped_mlp_body,
            grid_spec=grid_spec,
            out_shape=jax.ShapeDtypeStruct((cap_h, hidden // 2), jnp.int32),
        )(block_meta[h], x_sorted, gate_up_proj, down_proj)
        out_a, out_b = _sc_dual_row_gather(out_sorted, pos2[:, 0],
                                           pos2[:, 1])
        gathered.append((out_a, out_b))

    pair_block = 512
    npb = n_h // pair_block
    final = pl.pallas_call(
        _pair_add_body,
        grid=(N_PIPE, npb),
        in_specs=[
            pl.BlockSpec((pair_block, hidden // 2), lambda h, i: (i, 0)),
            pl.BlockSpec((pair_block, hidden // 2), lambda h, i: (i, 0)),
            pl.BlockSpec((pair_block, hidden // 2), lambda h, i: (i, 0)),
            pl.BlockSpec((pair_block, hidden // 2), lambda h, i: (i, 0)),
            pl.BlockSpec((pair_block, TOP_K),
                         lambda h, i: (h * (n_h // pair_block) + i, 0)),
        ],
        out_specs=pl.BlockSpec((pair_block, hidden),
                               lambda h, i: (h * (n_h // pair_block) + i, 0)),
        out_shape=jax.ShapeDtypeStruct((n_tok, hidden), jnp.float32),
    )(gathered[0][0], gathered[0][1], gathered[1][0], gathered[1][1],
      w_pair.reshape(n_tok, TOP_K))

    return final.reshape(b, s, hidden)


# speedup vs baseline: 1.8048x; 1.8048x over previous
import functools

import jax
import jax.numpy as jnp
from jax import lax
from jax.experimental import pallas as pl
from jax.experimental.pallas import tpu as pltpu
from jax.experimental.pallas import tpu_sc as plsc

NUM_EXPERTS = 8
TOP_K = 2
T_BLOCK = 256
SC_CHUNK = 16
NC, NS = 2, 16
NW = NC * NS


def _sc_row_gather(table, idx, n_rows):
    d = table.shape[1]
    dtype = table.dtype
    rows_per_w = n_rows // NW
    n_chunks = rows_per_w // SC_CHUNK
    mesh = plsc.VectorSubcoreMesh(core_axis_name="c", subcore_axis_name="s")

    @functools.partial(
        pl.kernel,
        out_type=jax.ShapeDtypeStruct((n_rows, d), dtype),
        mesh=mesh,
        scratch_types=[
            pltpu.VMEM((SC_CHUNK,), jnp.int32),
            pltpu.VMEM((SC_CHUNK,), jnp.int32),
            pltpu.VMEM((SC_CHUNK, d), dtype),
            pltpu.VMEM((SC_CHUNK, d), dtype),
            pltpu.SemaphoreType.DMA,
            pltpu.SemaphoreType.DMA,
            pltpu.SemaphoreType.DMA,
            pltpu.SemaphoreType.DMA,
        ],
    )
    def gather_kernel(table_hbm, idx_hbm, out_hbm, idx_v0, idx_v1, rows_v0,
                      rows_v1, g_sem0, g_sem1, w_sem0, w_sem1):
        wid = lax.axis_index("s") * NC + lax.axis_index("c")

        @pl.loop(0, n_chunks, step=2)
        def _(c):
            base0 = wid * rows_per_w + c * SC_CHUNK
            base1 = base0 + SC_CHUNK
            pltpu.sync_copy(idx_hbm.at[pl.ds(base0, SC_CHUNK)], idx_v0)
            g0 = pltpu.async_copy(table_hbm.at[idx_v0], rows_v0, g_sem0)
            pltpu.sync_copy(idx_hbm.at[pl.ds(base1, SC_CHUNK)], idx_v1)
            g1 = pltpu.async_copy(table_hbm.at[idx_v1], rows_v1, g_sem1)
            g0.wait()
            w0 = pltpu.async_copy(rows_v0, out_hbm.at[pl.ds(base0, SC_CHUNK)],
                                  w_sem0)
            g1.wait()
            w1 = pltpu.async_copy(rows_v1, out_hbm.at[pl.ds(base1, SC_CHUNK)],
                                  w_sem1)
            w0.wait()
            w1.wait()

    return gather_kernel(table, idx)


def _sc_dual_row_gather(table, idx_a, idx_b):
    d = table.shape[1]
    dtype = table.dtype
    n_rows = idx_a.shape[0]
    rows_per_w = n_rows // NW
    n_chunks = rows_per_w // SC_CHUNK
    mesh = plsc.VectorSubcoreMesh(core_axis_name="c", subcore_axis_name="s")

    @functools.partial(
        pl.kernel,
        out_type=(jax.ShapeDtypeStruct((n_rows, d), dtype),
                  jax.ShapeDtypeStruct((n_rows, d), dtype)),
        mesh=mesh,
        scratch_types=[
            pltpu.VMEM((SC_CHUNK,), jnp.int32),
            pltpu.VMEM((SC_CHUNK,), jnp.int32),
            pltpu.VMEM((SC_CHUNK, d), dtype),
            pltpu.VMEM((SC_CHUNK, d), dtype),
            pltpu.SemaphoreType.DMA,
            pltpu.SemaphoreType.DMA,
            pltpu.SemaphoreType.DMA,
            pltpu.SemaphoreType.DMA,
        ],
    )
    def dual_gather_kernel(table_hbm, ia_hbm, ib_hbm, outa_hbm, outb_hbm,
                           ia_v, ib_v, rows_a, rows_b, ga_sem, gb_sem,
                           wa_sem, wb_sem):
        wid = lax.axis_index("s") * NC + lax.axis_index("c")

        @pl.loop(0, n_chunks)
        def _(c):
            base = wid * rows_per_w + c * SC_CHUNK
            pltpu.sync_copy(ia_hbm.at[pl.ds(base, SC_CHUNK)], ia_v)
            ga = pltpu.async_copy(table_hbm.at[ia_v], rows_a, ga_sem)
            pltpu.sync_copy(ib_hbm.at[pl.ds(base, SC_CHUNK)], ib_v)
            gb = pltpu.async_copy(table_hbm.at[ib_v], rows_b, gb_sem)
            ga.wait()
            wa = pltpu.async_copy(rows_a, outa_hbm.at[pl.ds(base, SC_CHUNK)],
                                  wa_sem)
            gb.wait()
            wb = pltpu.async_copy(rows_b, outb_hbm.at[pl.ds(base, SC_CHUNK)],
                                  wb_sem)
            wa.wait()
            wb.wait()

    return dual_gather_kernel(table, idx_a, idx_b)


def _sc_dispatch_scatter(hs, pos_a, pos_b, capacity):
    n_tok, d = hs.shape
    rows_per_w = n_tok // NW
    n_chunks = rows_per_w // SC_CHUNK
    mesh = plsc.VectorSubcoreMesh(core_axis_name="c", subcore_axis_name="s")

    @functools.partial(
        pl.kernel,
        out_type=jax.ShapeDtypeStruct((capacity, d), hs.dtype),
        mesh=mesh,
        scratch_types=[
            pltpu.VMEM((SC_CHUNK,), jnp.int32),
            pltpu.VMEM((SC_CHUNK,), jnp.int32),
            pltpu.VMEM((SC_CHUNK,), jnp.int32),
            pltpu.VMEM((SC_CHUNK,), jnp.int32),
            pltpu.VMEM((SC_CHUNK, d), hs.dtype),
            pltpu.VMEM((SC_CHUNK, d), hs.dtype),
            pltpu.SemaphoreType.DMA,
            pltpu.SemaphoreType.DMA,
            pltpu.SemaphoreType.DMA,
            pltpu.SemaphoreType.DMA,
            pltpu.SemaphoreType.DMA,
            pltpu.SemaphoreType.DMA,
        ],
    )
    def scatter_kernel(hs_hbm, pa_hbm, pb_hbm, out_hbm, ia0, ib0, ia1, ib1,
                       rows0, rows1, r_sem0, r_sem1, a_sem0, a_sem1, b_sem0,
                       b_sem1):
        wid = lax.axis_index("s") * NC + lax.axis_index("c")
        idx_a = (ia0, ia1)
        idx_b = (ib0, ib1)
        rows = (rows0, rows1)
        r_sem = (r_sem0, r_sem1)
        a_sem = (a_sem0, a_sem1)
        b_sem = (b_sem0, b_sem1)

        @pl.loop(0, n_chunks, step=2)
        def _(c):
            scats = []
            for k in range(2):
                base = wid * rows_per_w + c * SC_CHUNK + k * SC_CHUNK
                r = pltpu.async_copy(hs_hbm.at[pl.ds(base, SC_CHUNK)],
                                     rows[k], r_sem[k])
                pltpu.sync_copy(pa_hbm.at[pl.ds(base, SC_CHUNK)], idx_a[k])
                pltpu.sync_copy(pb_hbm.at[pl.ds(base, SC_CHUNK)], idx_b[k])
                r.wait()
                scats.append(pltpu.async_copy(rows[k], out_hbm.at[idx_a[k]],
                                              a_sem[k]))
                scats.append(pltpu.async_copy(rows[k], out_hbm.at[idx_b[k]],
                                              b_sem[k]))
            for cp in scats:
                cp.wait()

    return scatter_kernel(hs, pos_a, pos_b)


def _grouped_mlp_body(meta_ref, x_ref, wgu_ref, wd_ref, out_ref):
    b = pl.program_id(0)
    e = meta_ref[b]

    @pl.when(e < NUM_EXPERTS)
    def _compute():
        inter = wd_ref.shape[1]
        x = x_ref[...].astype(jnp.bfloat16)
        gu = jnp.dot(x, wgu_ref[0].astype(jnp.bfloat16),
                     preferred_element_type=jnp.float32)
        gate = gu[:, :inter]
        up = gu[:, inter:]
        h = (gate * jax.nn.sigmoid(gate)) * up
        out = jnp.dot(h.astype(jnp.bfloat16), wd_ref[0].astype(jnp.bfloat16),
                      preferred_element_type=jnp.float32)
        half = out.shape[1] // 2
        lo = jax.lax.bitcast_convert_type(
            out[:, :half].astype(jnp.bfloat16), jnp.uint16).astype(jnp.uint32)
        hi = jax.lax.bitcast_convert_type(
            out[:, half:].astype(jnp.bfloat16), jnp.uint16).astype(jnp.uint32)
        out_ref[...] = jax.lax.bitcast_convert_type(lo | (hi << 16),
                                                    jnp.int32)


def _unpack_bf16_pair(p):
    u = jax.lax.bitcast_convert_type(p, jnp.uint32)
    lo = jax.lax.bitcast_convert_type((u & 0xFFFF).astype(jnp.uint16),
                                      jnp.bfloat16)
    hi = jax.lax.bitcast_convert_type((u >> 16).astype(jnp.uint16),
                                      jnp.bfloat16)
    return lo.astype(jnp.float32), hi.astype(jnp.float32)


def _pair_add_body(a_ref, b_ref, w_ref, out_ref):
    w = w_ref[...]
    half = out_ref.shape[1] // 2
    a_lo, a_hi = _unpack_bf16_pair(a_ref[...])
    b_lo, b_hi = _unpack_bf16_pair(b_ref[...])
    out_ref[:, :half] = a_lo * w[:, 0:1] + b_lo * w[:, 1:2]
    out_ref[:, half:] = a_hi * w[:, 0:1] + b_hi * w[:, 1:2]


def kernel(hidden_states, routing_weights, router_indices, gate_up_proj,
           down_proj):
    b, s, hidden = hidden_states.shape
    n_tok = b * s
    num_experts, _, two_inter = gate_up_proj.shape
    inter = two_inter // 2
    n_assign = n_tok * TOP_K
    capacity = n_assign + num_experts * T_BLOCK
    nb = capacity // T_BLOCK

    hs = hidden_states.reshape(n_tok, hidden)
    rw = routing_weights.reshape(n_tok, num_experts)
    ri = router_indices.reshape(n_tok, TOP_K).astype(jnp.int32)

    eids = jnp.arange(num_experts, dtype=jnp.int32)
    sel = ri[:, :, None] == eids[None, None, :]
    w_pair = jnp.sum(jnp.where(sel, rw[:, None, :], 0.0), axis=2)
    dup = ri[:, 1] == ri[:, 0]
    w_pair = jnp.concatenate(
        [w_pair[:, 0:1], jnp.where(dup[:, None], 0.0, w_pair[:, 1:2])], axis=1)

    onehot = sel.reshape(n_assign, num_experts)
    csum = jnp.cumsum(onehot.astype(jnp.int32), axis=0)
    counts = csum[-1]
    rank_a = jnp.sum(jnp.where(onehot, csum, 0), axis=1) - 1
    padded_counts = ((counts + T_BLOCK - 1) // T_BLOCK) * T_BLOCK
    padded_starts = jnp.concatenate(
        [jnp.zeros((1,), jnp.int32),
         jnp.cumsum(padded_counts)[:-1].astype(jnp.int32)])
    start_a = jnp.sum(jnp.where(onehot, padded_starts[None, :], 0), axis=1)
    pos_a = (start_a + rank_a).astype(jnp.int32)
    padded_total = jnp.sum(padded_counts)
    blk_rows = jnp.arange(nb, dtype=jnp.int32) * T_BLOCK
    be = jnp.sum((padded_starts[None, :] <= blk_rows[:, None]).astype(
        jnp.int32), axis=1) - 1
    block_meta = jnp.where(blk_rows < padded_total, be,
                           num_experts).astype(jnp.int32)

    pos_2 = pos_a.reshape(n_tok, TOP_K)
    x_sorted = _sc_dispatch_scatter(hs, pos_2[:, 0], pos_2[:, 1],
                                    capacity)

    grid_spec = pltpu.PrefetchScalarGridSpec(
        num_scalar_prefetch=1,
        grid=(nb,),
        in_specs=[
            pl.BlockSpec((T_BLOCK, hidden), lambda i, m: (i, 0)),
            pl.BlockSpec((1, hidden, two_inter),
                         lambda i, m: (jnp.minimum(m[i], NUM_EXPERTS - 1), 0, 0)),
            pl.BlockSpec((1, inter, hidden),
                         lambda i, m: (jnp.minimum(m[i], NUM_EXPERTS - 1), 0, 0)),
        ],
        out_specs=pl.BlockSpec((T_BLOCK, hidden // 2), lambda i, m: (i, 0)),
    )
    out_sorted = pl.pallas_call(
        _grouped_mlp_body,
        grid_spec=grid_spec,
        out_shape=jax.ShapeDtypeStruct((capacity, hidden // 2), jnp.int32),
    )(block_meta, x_sorted, gate_up_proj, down_proj)

    out_a, out_b = _sc_dual_row_gather(out_sorted, pos_2[:, 0], pos_2[:, 1])

    pair_block = 512
    final = pl.pallas_call(
        _pair_add_body,
        grid=(n_tok // pair_block,),
        in_specs=[
            pl.BlockSpec((pair_block, hidden // 2), lambda i: (i, 0)),
            pl.BlockSpec((pair_block, hidden // 2), lambda i: (i, 0)),
            pl.BlockSpec((pair_block, TOP_K), lambda i: (i, 0)),
        ],
        out_specs=pl.BlockSpec((pair_block, hidden), lambda i: (i, 0)),
        out_shape=jax.ShapeDtypeStruct((n_tok, hidden), jnp.float32),
    )(out_a, out_b, w_pair)

    return final.reshape(b, s, hidden)


# speedup vs baseline: 1.8252x; 1.0113x over previous
import functools

import jax
import jax.numpy as jnp
from jax import lax
from jax.experimental import pallas as pl
from jax.experimental.pallas import tpu as pltpu
from jax.experimental.pallas import tpu_sc as plsc

NUM_EXPERTS = 8
TOP_K = 2
T_BLOCK = 512
SC_CHUNK = 16
NC, NS = 2, 16
NW = NC * NS


def _sc_row_gather(table, idx, n_rows):
    d = table.shape[1]
    dtype = table.dtype
    rows_per_w = n_rows // NW
    n_chunks = rows_per_w // SC_CHUNK
    mesh = plsc.VectorSubcoreMesh(core_axis_name="c", subcore_axis_name="s")

    @functools.partial(
        pl.kernel,
        out_type=jax.ShapeDtypeStruct((n_rows, d), dtype),
        mesh=mesh,
        scratch_types=[
            pltpu.VMEM((SC_CHUNK,), jnp.int32),
            pltpu.VMEM((SC_CHUNK,), jnp.int32),
            pltpu.VMEM((SC_CHUNK, d), dtype),
            pltpu.VMEM((SC_CHUNK, d), dtype),
            pltpu.SemaphoreType.DMA,
            pltpu.SemaphoreType.DMA,
            pltpu.SemaphoreType.DMA,
            pltpu.SemaphoreType.DMA,
        ],
    )
    def gather_kernel(table_hbm, idx_hbm, out_hbm, idx_v0, idx_v1, rows_v0,
                      rows_v1, g_sem0, g_sem1, w_sem0, w_sem1):
        wid = lax.axis_index("s") * NC + lax.axis_index("c")

        @pl.loop(0, n_chunks, step=2)
        def _(c):
            base0 = wid * rows_per_w + c * SC_CHUNK
            base1 = base0 + SC_CHUNK
            pltpu.sync_copy(idx_hbm.at[pl.ds(base0, SC_CHUNK)], idx_v0)
            g0 = pltpu.async_copy(table_hbm.at[idx_v0], rows_v0, g_sem0)
            pltpu.sync_copy(idx_hbm.at[pl.ds(base1, SC_CHUNK)], idx_v1)
            g1 = pltpu.async_copy(table_hbm.at[idx_v1], rows_v1, g_sem1)
            g0.wait()
            w0 = pltpu.async_copy(rows_v0, out_hbm.at[pl.ds(base0, SC_CHUNK)],
                                  w_sem0)
            g1.wait()
            w1 = pltpu.async_copy(rows_v1, out_hbm.at[pl.ds(base1, SC_CHUNK)],
                                  w_sem1)
            w0.wait()
            w1.wait()

    return gather_kernel(table, idx)


def _sc_dual_row_gather(table, idx_a, idx_b):
    d = table.shape[1]
    dtype = table.dtype
    n_rows = idx_a.shape[0]
    rows_per_w = n_rows // NW
    n_chunks = rows_per_w // SC_CHUNK
    mesh = plsc.VectorSubcoreMesh(core_axis_name="c", subcore_axis_name="s")

    @functools.partial(
        pl.kernel,
        out_type=(jax.ShapeDtypeStruct((n_rows, d), dtype),
                  jax.ShapeDtypeStruct((n_rows, d), dtype)),
        mesh=mesh,
        scratch_types=[
            pltpu.VMEM((SC_CHUNK,), jnp.int32),
            pltpu.VMEM((SC_CHUNK,), jnp.int32),
            pltpu.VMEM((SC_CHUNK, d), dtype),
            pltpu.VMEM((SC_CHUNK, d), dtype),
            pltpu.SemaphoreType.DMA,
            pltpu.SemaphoreType.DMA,
            pltpu.SemaphoreType.DMA,
            pltpu.SemaphoreType.DMA,
        ],
    )
    def dual_gather_kernel(table_hbm, ia_hbm, ib_hbm, outa_hbm, outb_hbm,
                           ia_v, ib_v, rows_a, rows_b, ga_sem, gb_sem,
                           wa_sem, wb_sem):
        wid = lax.axis_index("s") * NC + lax.axis_index("c")

        @pl.loop(0, n_chunks)
        def _(c):
            base = wid * rows_per_w + c * SC_CHUNK
            pltpu.sync_copy(ia_hbm.at[pl.ds(base, SC_CHUNK)], ia_v)
            ga = pltpu.async_copy(table_hbm.at[ia_v], rows_a, ga_sem)
            pltpu.sync_copy(ib_hbm.at[pl.ds(base, SC_CHUNK)], ib_v)
            gb = pltpu.async_copy(table_hbm.at[ib_v], rows_b, gb_sem)
            ga.wait()
            wa = pltpu.async_copy(rows_a, outa_hbm.at[pl.ds(base, SC_CHUNK)],
                                  wa_sem)
            gb.wait()
            wb = pltpu.async_copy(rows_b, outb_hbm.at[pl.ds(base, SC_CHUNK)],
                                  wb_sem)
            wa.wait()
            wb.wait()

    return dual_gather_kernel(table, idx_a, idx_b)


def _sc_dispatch_scatter(hs, pos_a, pos_b, capacity):
    n_tok, d = hs.shape
    rows_per_w = n_tok // NW
    n_chunks = rows_per_w // SC_CHUNK
    mesh = plsc.VectorSubcoreMesh(core_axis_name="c", subcore_axis_name="s")

    @functools.partial(
        pl.kernel,
        out_type=jax.ShapeDtypeStruct((capacity, d), hs.dtype),
        mesh=mesh,
        scratch_types=[
            pltpu.VMEM((SC_CHUNK,), jnp.int32),
            pltpu.VMEM((SC_CHUNK,), jnp.int32),
            pltpu.VMEM((SC_CHUNK,), jnp.int32),
            pltpu.VMEM((SC_CHUNK,), jnp.int32),
            pltpu.VMEM((SC_CHUNK, d), hs.dtype),
            pltpu.VMEM((SC_CHUNK, d), hs.dtype),
            pltpu.SemaphoreType.DMA,
            pltpu.SemaphoreType.DMA,
            pltpu.SemaphoreType.DMA,
            pltpu.SemaphoreType.DMA,
            pltpu.SemaphoreType.DMA,
            pltpu.SemaphoreType.DMA,
        ],
    )
    def scatter_kernel(hs_hbm, pa_hbm, pb_hbm, out_hbm, ia0, ib0, ia1, ib1,
                       rows0, rows1, r_sem0, r_sem1, a_sem0, a_sem1, b_sem0,
                       b_sem1):
        wid = lax.axis_index("s") * NC + lax.axis_index("c")
        idx_a = (ia0, ia1)
        idx_b = (ib0, ib1)
        rows = (rows0, rows1)
        r_sem = (r_sem0, r_sem1)
        a_sem = (a_sem0, a_sem1)
        b_sem = (b_sem0, b_sem1)

        @pl.loop(0, n_chunks, step=2)
        def _(c):
            scats = []
            for k in range(2):
                base = wid * rows_per_w + c * SC_CHUNK + k * SC_CHUNK
                r = pltpu.async_copy(hs_hbm.at[pl.ds(base, SC_CHUNK)],
                                     rows[k], r_sem[k])
                pltpu.sync_copy(pa_hbm.at[pl.ds(base, SC_CHUNK)], idx_a[k])
                pltpu.sync_copy(pb_hbm.at[pl.ds(base, SC_CHUNK)], idx_b[k])
                r.wait()
                scats.append(pltpu.async_copy(rows[k], out_hbm.at[idx_a[k]],
                                              a_sem[k]))
                scats.append(pltpu.async_copy(rows[k], out_hbm.at[idx_b[k]],
                                              b_sem[k]))
            for cp in scats:
                cp.wait()

    return scatter_kernel(hs, pos_a, pos_b)


def _grouped_mlp_body(meta_ref, x_ref, wgu_ref, wd_ref, out_ref):
    b = pl.program_id(0)
    e = meta_ref[b]

    @pl.when(e < NUM_EXPERTS)
    def _compute():
        inter = wd_ref.shape[1]
        x = x_ref[...].astype(jnp.bfloat16)
        gu = jnp.dot(x, wgu_ref[0].astype(jnp.bfloat16),
                     preferred_element_type=jnp.float32)
        gate = gu[:, :inter]
        up = gu[:, inter:]
        h = (gate * jax.nn.sigmoid(gate)) * up
        out = jnp.dot(h.astype(jnp.bfloat16), wd_ref[0].astype(jnp.bfloat16),
                      preferred_element_type=jnp.float32)
        half = out.shape[1] // 2
        lo = jax.lax.bitcast_convert_type(
            out[:, :half].astype(jnp.bfloat16), jnp.uint16).astype(jnp.uint32)
        hi = jax.lax.bitcast_convert_type(
            out[:, half:].astype(jnp.bfloat16), jnp.uint16).astype(jnp.uint32)
        out_ref[...] = jax.lax.bitcast_convert_type(lo | (hi << 16),
                                                    jnp.int32)


def _unpack_bf16_pair(p):
    u = jax.lax.bitcast_convert_type(p, jnp.uint32)
    lo = jax.lax.bitcast_convert_type((u & 0xFFFF).astype(jnp.uint16),
                                      jnp.bfloat16)
    hi = jax.lax.bitcast_convert_type((u >> 16).astype(jnp.uint16),
                                      jnp.bfloat16)
    return lo.astype(jnp.float32), hi.astype(jnp.float32)


def _pair_add_body(a_ref, b_ref, w_ref, out_ref):
    w = w_ref[...]
    half = out_ref.shape[1] // 2
    a_lo, a_hi = _unpack_bf16_pair(a_ref[...])
    b_lo, b_hi = _unpack_bf16_pair(b_ref[...])
    out_ref[:, :half] = a_lo * w[:, 0:1] + b_lo * w[:, 1:2]
    out_ref[:, half:] = a_hi * w[:, 0:1] + b_hi * w[:, 1:2]


def kernel(hidden_states, routing_weights, router_indices, gate_up_proj,
           down_proj):
    b, s, hidden = hidden_states.shape
    n_tok = b * s
    num_experts, _, two_inter = gate_up_proj.shape
    inter = two_inter // 2
    n_assign = n_tok * TOP_K
    capacity = n_assign + num_experts * T_BLOCK
    nb = capacity // T_BLOCK

    hs = hidden_states.reshape(n_tok, hidden)
    rw = routing_weights.reshape(n_tok, num_experts)
    ri = router_indices.reshape(n_tok, TOP_K).astype(jnp.int32)

    eids = jnp.arange(num_experts, dtype=jnp.int32)
    sel = ri[:, :, None] == eids[None, None, :]
    w_pair = jnp.sum(jnp.where(sel, rw[:, None, :], 0.0), axis=2)
    dup = ri[:, 1] == ri[:, 0]
    w_pair = jnp.concatenate(
        [w_pair[:, 0:1], jnp.where(dup[:, None], 0.0, w_pair[:, 1:2])], axis=1)

    onehot = sel.reshape(n_assign, num_experts)
    csum = jnp.cumsum(onehot.astype(jnp.int32), axis=0)
    counts = csum[-1]
    rank_a = jnp.sum(jnp.where(onehot, csum, 0), axis=1) - 1
    padded_counts = ((counts + T_BLOCK - 1) // T_BLOCK) * T_BLOCK
    padded_starts = jnp.concatenate(
        [jnp.zeros((1,), jnp.int32),
         jnp.cumsum(padded_counts)[:-1].astype(jnp.int32)])
    start_a = jnp.sum(jnp.where(onehot, padded_starts[None, :], 0), axis=1)
    pos_a = (start_a + rank_a).astype(jnp.int32)
    padded_total = jnp.sum(padded_counts)
    blk_rows = jnp.arange(nb, dtype=jnp.int32) * T_BLOCK
    be = jnp.sum((padded_starts[None, :] <= blk_rows[:, None]).astype(
        jnp.int32), axis=1) - 1
    block_meta = jnp.where(blk_rows < padded_total, be,
                           num_experts).astype(jnp.int32)

    pos_2 = pos_a.reshape(n_tok, TOP_K)
    x_sorted = _sc_dispatch_scatter(hs, pos_2[:, 0], pos_2[:, 1],
                                    capacity)

    grid_spec = pltpu.PrefetchScalarGridSpec(
        num_scalar_prefetch=1,
        grid=(nb,),
        in_specs=[
            pl.BlockSpec((T_BLOCK, hidden), lambda i, m: (i, 0)),
            pl.BlockSpec((1, hidden, two_inter),
                         lambda i, m: (jnp.minimum(m[i], NUM_EXPERTS - 1), 0, 0)),
            pl.BlockSpec((1, inter, hidden),
                         lambda i, m: (jnp.minimum(m[i], NUM_EXPERTS - 1), 0, 0)),
        ],
        out_specs=pl.BlockSpec((T_BLOCK, hidden // 2), lambda i, m: (i, 0)),
    )
    out_sorted = pl.pallas_call(
        _grouped_mlp_body,
        grid_spec=grid_spec,
        out_shape=jax.ShapeDtypeStruct((capacity, hidden // 2), jnp.int32),
    )(block_meta, x_sorted, gate_up_proj, down_proj)

    out_a, out_b = _sc_dual_row_gather(out_sorted, pos_2[:, 0], pos_2[:, 1])

    pair_block = 512
    final = pl.pallas_call(
        _pair_add_body,
        grid=(n_tok // pair_block,),
        in_specs=[
            pl.BlockSpec((pair_block, hidden // 2), lambda i: (i, 0)),
            pl.BlockSpec((pair_block, hidden // 2), lambda i: (i, 0)),
            pl.BlockSpec((pair_block, TOP_K), lambda i: (i, 0)),
        ],
        out_specs=pl.BlockSpec((pair_block, hidden), lambda i: (i, 0)),
        out_shape=jax.ShapeDtypeStruct((n_tok, hidden), jnp.float32),
    )(out_a, out_b, w_pair)

    return final.reshape(b, s, hidden)


# speedup vs baseline: 1.8448x; 1.0107x over previous
import functools

import jax
import jax.numpy as jnp
from jax import lax
from jax.experimental import pallas as pl
from jax.experimental.pallas import tpu as pltpu
from jax.experimental.pallas import tpu_sc as plsc

NUM_EXPERTS = 8
TOP_K = 2
T_BLOCK = 512
SC_CHUNK = 16
NC, NS = 2, 16
NW = NC * NS


def _sc_row_gather(table, idx, n_rows):
    d = table.shape[1]
    dtype = table.dtype
    rows_per_w = n_rows // NW
    n_chunks = rows_per_w // SC_CHUNK
    mesh = plsc.VectorSubcoreMesh(core_axis_name="c", subcore_axis_name="s")

    @functools.partial(
        pl.kernel,
        out_type=jax.ShapeDtypeStruct((n_rows, d), dtype),
        mesh=mesh,
        scratch_types=[
            pltpu.VMEM((SC_CHUNK,), jnp.int32),
            pltpu.VMEM((SC_CHUNK,), jnp.int32),
            pltpu.VMEM((SC_CHUNK, d), dtype),
            pltpu.VMEM((SC_CHUNK, d), dtype),
            pltpu.SemaphoreType.DMA,
            pltpu.SemaphoreType.DMA,
            pltpu.SemaphoreType.DMA,
            pltpu.SemaphoreType.DMA,
        ],
    )
    def gather_kernel(table_hbm, idx_hbm, out_hbm, idx_v0, idx_v1, rows_v0,
                      rows_v1, g_sem0, g_sem1, w_sem0, w_sem1):
        wid = lax.axis_index("s") * NC + lax.axis_index("c")

        @pl.loop(0, n_chunks, step=2)
        def _(c):
            base0 = wid * rows_per_w + c * SC_CHUNK
            base1 = base0 + SC_CHUNK
            pltpu.sync_copy(idx_hbm.at[pl.ds(base0, SC_CHUNK)], idx_v0)
            g0 = pltpu.async_copy(table_hbm.at[idx_v0], rows_v0, g_sem0)
            pltpu.sync_copy(idx_hbm.at[pl.ds(base1, SC_CHUNK)], idx_v1)
            g1 = pltpu.async_copy(table_hbm.at[idx_v1], rows_v1, g_sem1)
            g0.wait()
            w0 = pltpu.async_copy(rows_v0, out_hbm.at[pl.ds(base0, SC_CHUNK)],
                                  w_sem0)
            g1.wait()
            w1 = pltpu.async_copy(rows_v1, out_hbm.at[pl.ds(base1, SC_CHUNK)],
                                  w_sem1)
            w0.wait()
            w1.wait()

    return gather_kernel(table, idx)


def _sc_dual_row_gather(table, idx_a, idx_b, chunk=SC_CHUNK):
    d = table.shape[1]
    dtype = table.dtype
    n_rows = idx_a.shape[0]
    rows_per_w = n_rows // NW
    n_chunks = rows_per_w // chunk
    mesh = plsc.VectorSubcoreMesh(core_axis_name="c", subcore_axis_name="s")

    @functools.partial(
        pl.kernel,
        out_type=(jax.ShapeDtypeStruct((n_rows, d), dtype),
                  jax.ShapeDtypeStruct((n_rows, d), dtype)),
        mesh=mesh,
        scratch_types=[
            pltpu.VMEM((chunk,), jnp.int32),
            pltpu.VMEM((chunk,), jnp.int32),
            pltpu.VMEM((chunk, d), dtype),
            pltpu.VMEM((chunk, d), dtype),
            pltpu.SemaphoreType.DMA,
            pltpu.SemaphoreType.DMA,
            pltpu.SemaphoreType.DMA,
            pltpu.SemaphoreType.DMA,
        ],
    )
    def dual_gather_kernel(table_hbm, ia_hbm, ib_hbm, outa_hbm, outb_hbm,
                           ia_v, ib_v, rows_a, rows_b, ga_sem, gb_sem,
                           wa_sem, wb_sem):
        wid = lax.axis_index("s") * NC + lax.axis_index("c")

        @pl.loop(0, n_chunks)
        def _(c):
            base = wid * rows_per_w + c * chunk
            pltpu.sync_copy(ia_hbm.at[pl.ds(base, chunk)], ia_v)
            ga = pltpu.async_copy(table_hbm.at[ia_v], rows_a, ga_sem)
            pltpu.sync_copy(ib_hbm.at[pl.ds(base, chunk)], ib_v)
            gb = pltpu.async_copy(table_hbm.at[ib_v], rows_b, gb_sem)
            ga.wait()
            wa = pltpu.async_copy(rows_a, outa_hbm.at[pl.ds(base, chunk)],
                                  wa_sem)
            gb.wait()
            wb = pltpu.async_copy(rows_b, outb_hbm.at[pl.ds(base, chunk)],
                                  wb_sem)
            wa.wait()
            wb.wait()

    return dual_gather_kernel(table, idx_a, idx_b)


def _sc_dispatch_scatter(hs, pos_a, pos_b, capacity):
    n_tok, d = hs.shape
    rows_per_w = n_tok // NW
    n_chunks = rows_per_w // SC_CHUNK
    mesh = plsc.VectorSubcoreMesh(core_axis_name="c", subcore_axis_name="s")

    @functools.partial(
        pl.kernel,
        out_type=jax.ShapeDtypeStruct((capacity, d), hs.dtype),
        mesh=mesh,
        scratch_types=[
            pltpu.VMEM((SC_CHUNK,), jnp.int32),
            pltpu.VMEM((SC_CHUNK,), jnp.int32),
            pltpu.VMEM((SC_CHUNK,), jnp.int32),
            pltpu.VMEM((SC_CHUNK,), jnp.int32),
            pltpu.VMEM((SC_CHUNK, d), hs.dtype),
            pltpu.VMEM((SC_CHUNK, d), hs.dtype),
            pltpu.SemaphoreType.DMA,
            pltpu.SemaphoreType.DMA,
            pltpu.SemaphoreType.DMA,
            pltpu.SemaphoreType.DMA,
            pltpu.SemaphoreType.DMA,
            pltpu.SemaphoreType.DMA,
        ],
    )
    def scatter_kernel(hs_hbm, pa_hbm, pb_hbm, out_hbm, ia0, ib0, ia1, ib1,
                       rows0, rows1, r_sem0, r_sem1, a_sem0, a_sem1, b_sem0,
                       b_sem1):
        wid = lax.axis_index("s") * NC + lax.axis_index("c")
        idx_a = (ia0, ia1)
        idx_b = (ib0, ib1)
        rows = (rows0, rows1)
        r_sem = (r_sem0, r_sem1)
        a_sem = (a_sem0, a_sem1)
        b_sem = (b_sem0, b_sem1)

        @pl.loop(0, n_chunks, step=2)
        def _(c):
            scats = []
            for k in range(2):
                base = wid * rows_per_w + c * SC_CHUNK + k * SC_CHUNK
                r = pltpu.async_copy(hs_hbm.at[pl.ds(base, SC_CHUNK)],
                                     rows[k], r_sem[k])
                pltpu.sync_copy(pa_hbm.at[pl.ds(base, SC_CHUNK)], idx_a[k])
                pltpu.sync_copy(pb_hbm.at[pl.ds(base, SC_CHUNK)], idx_b[k])
                r.wait()
                scats.append(pltpu.async_copy(rows[k], out_hbm.at[idx_a[k]],
                                              a_sem[k]))
                scats.append(pltpu.async_copy(rows[k], out_hbm.at[idx_b[k]],
                                              b_sem[k]))
            for cp in scats:
                cp.wait()

    return scatter_kernel(hs, pos_a, pos_b)


def _grouped_mlp_body(meta_ref, x_ref, wgu_ref, wd_ref, out_ref):
    b = pl.program_id(0)
    e = meta_ref[b]

    @pl.when(e < NUM_EXPERTS)
    def _compute():
        inter = wd_ref.shape[1]
        x = x_ref[...].astype(jnp.bfloat16)
        gu = jnp.dot(x, wgu_ref[0].astype(jnp.bfloat16),
                     preferred_element_type=jnp.float32)
        gate = gu[:, :inter]
        up = gu[:, inter:]
        h = (gate * jax.nn.sigmoid(gate)) * up
        out = jnp.dot(h.astype(jnp.bfloat16), wd_ref[0].astype(jnp.bfloat16),
                      preferred_element_type=jnp.float32)
        half = out.shape[1] // 2
        lo = jax.lax.bitcast_convert_type(
            out[:, :half].astype(jnp.bfloat16), jnp.uint16).astype(jnp.uint32)
        hi = jax.lax.bitcast_convert_type(
            out[:, half:].astype(jnp.bfloat16), jnp.uint16).astype(jnp.uint32)
        out_ref[...] = jax.lax.bitcast_convert_type(lo | (hi << 16),
                                                    jnp.int32)


def _unpack_bf16_pair(p):
    u = jax.lax.bitcast_convert_type(p, jnp.uint32)
    lo = jax.lax.bitcast_convert_type((u & 0xFFFF).astype(jnp.uint16),
                                      jnp.bfloat16)
    hi = jax.lax.bitcast_convert_type((u >> 16).astype(jnp.uint16),
                                      jnp.bfloat16)
    return lo.astype(jnp.float32), hi.astype(jnp.float32)


def _pair_add_body(a_ref, b_ref, w_ref, out_ref):
    w = w_ref[...]
    half = out_ref.shape[1] // 2
    a_lo, a_hi = _unpack_bf16_pair(a_ref[...])
    b_lo, b_hi = _unpack_bf16_pair(b_ref[...])
    out_ref[:, :half] = a_lo * w[:, 0:1] + b_lo * w[:, 1:2]
    out_ref[:, half:] = a_hi * w[:, 0:1] + b_hi * w[:, 1:2]


def kernel(hidden_states, routing_weights, router_indices, gate_up_proj,
           down_proj):
    b, s, hidden = hidden_states.shape
    n_tok = b * s
    num_experts, _, two_inter = gate_up_proj.shape
    inter = two_inter // 2
    n_assign = n_tok * TOP_K
    capacity = n_assign + num_experts * T_BLOCK
    nb = capacity // T_BLOCK

    hs = hidden_states.reshape(n_tok, hidden)
    rw = routing_weights.reshape(n_tok, num_experts)
    ri = router_indices.reshape(n_tok, TOP_K).astype(jnp.int32)

    eids = jnp.arange(num_experts, dtype=jnp.int32)
    sel = ri[:, :, None] == eids[None, None, :]
    w_pair = jnp.sum(jnp.where(sel, rw[:, None, :], 0.0), axis=2)
    dup = ri[:, 1] == ri[:, 0]
    w_pair = jnp.concatenate(
        [w_pair[:, 0:1], jnp.where(dup[:, None], 0.0, w_pair[:, 1:2])], axis=1)

    onehot = sel.reshape(n_assign, num_experts)
    csum = jnp.cumsum(onehot.astype(jnp.int32), axis=0)
    counts = csum[-1]
    rank_a = jnp.sum(jnp.where(onehot, csum, 0), axis=1) - 1
    padded_counts = ((counts + T_BLOCK - 1) // T_BLOCK) * T_BLOCK
    padded_starts = jnp.concatenate(
        [jnp.zeros((1,), jnp.int32),
         jnp.cumsum(padded_counts)[:-1].astype(jnp.int32)])
    start_a = jnp.sum(jnp.where(onehot, padded_starts[None, :], 0), axis=1)
    pos_a = (start_a + rank_a).astype(jnp.int32)
    padded_total = jnp.sum(padded_counts)
    blk_rows = jnp.arange(nb, dtype=jnp.int32) * T_BLOCK
    be = jnp.sum((padded_starts[None, :] <= blk_rows[:, None]).astype(
        jnp.int32), axis=1) - 1
    block_meta = jnp.where(blk_rows < padded_total, be,
                           num_experts).astype(jnp.int32)

    pos_2 = pos_a.reshape(n_tok, TOP_K)
    x_sorted = _sc_dispatch_scatter(hs, pos_2[:, 0], pos_2[:, 1],
                                    capacity)

    grid_spec = pltpu.PrefetchScalarGridSpec(
        num_scalar_prefetch=1,
        grid=(nb,),
        in_specs=[
            pl.BlockSpec((T_BLOCK, hidden), lambda i, m: (i, 0)),
            pl.BlockSpec((1, hidden, two_inter),
                         lambda i, m: (jnp.minimum(m[i], NUM_EXPERTS - 1), 0, 0)),
            pl.BlockSpec((1, inter, hidden),
                         lambda i, m: (jnp.minimum(m[i], NUM_EXPERTS - 1), 0, 0)),
        ],
        out_specs=pl.BlockSpec((T_BLOCK, hidden // 2), lambda i, m: (i, 0)),
    )
    out_sorted = pl.pallas_call(
        _grouped_mlp_body,
        grid_spec=grid_spec,
        out_shape=jax.ShapeDtypeStruct((capacity, hidden // 2), jnp.int32),
    )(block_meta, x_sorted, gate_up_proj, down_proj)

    out_a, out_b = _sc_dual_row_gather(out_sorted, pos_2[:, 0], pos_2[:, 1],
                                       chunk=32)

    pair_block = 512
    final = pl.pallas_call(
        _pair_add_body,
        grid=(n_tok // pair_block,),
        in_specs=[
            pl.BlockSpec((pair_block, hidden // 2), lambda i: (i, 0)),
            pl.BlockSpec((pair_block, hidden // 2), lambda i: (i, 0)),
            pl.BlockSpec((pair_block, TOP_K), lambda i: (i, 0)),
        ],
        out_specs=pl.BlockSpec((pair_block, hidden), lambda i: (i, 0)),
        out_shape=jax.ShapeDtypeStruct((n_tok, hidden), jnp.float32),
    )(out_a, out_b, w_pair)

    return final.reshape(b, s, hidden)


# speedup vs baseline: 1.8903x; 1.0247x over previous
import functools

import jax
import jax.numpy as jnp
from jax import lax
from jax.experimental import pallas as pl
from jax.experimental.pallas import tpu as pltpu
from jax.experimental.pallas import tpu_sc as plsc

NUM_EXPERTS = 8
TOP_K = 2
T_BLOCK = 512
SC_CHUNK = 16
NC, NS = 2, 16
NW = NC * NS


def _sc_row_gather(table, idx, n_rows):
    d = table.shape[1]
    dtype = table.dtype
    rows_per_w = n_rows // NW
    n_chunks = rows_per_w // SC_CHUNK
    mesh = plsc.VectorSubcoreMesh(core_axis_name="c", subcore_axis_name="s")

    @functools.partial(
        pl.kernel,
        out_type=jax.ShapeDtypeStruct((n_rows, d), dtype),
        mesh=mesh,
        scratch_types=[
            pltpu.VMEM((SC_CHUNK,), jnp.int32),
            pltpu.VMEM((SC_CHUNK,), jnp.int32),
            pltpu.VMEM((SC_CHUNK, d), dtype),
            pltpu.VMEM((SC_CHUNK, d), dtype),
            pltpu.SemaphoreType.DMA,
            pltpu.SemaphoreType.DMA,
            pltpu.SemaphoreType.DMA,
            pltpu.SemaphoreType.DMA,
        ],
    )
    def gather_kernel(table_hbm, idx_hbm, out_hbm, idx_v0, idx_v1, rows_v0,
                      rows_v1, g_sem0, g_sem1, w_sem0, w_sem1):
        wid = lax.axis_index("s") * NC + lax.axis_index("c")

        @pl.loop(0, n_chunks, step=2)
        def _(c):
            base0 = wid * rows_per_w + c * SC_CHUNK
            base1 = base0 + SC_CHUNK
            pltpu.sync_copy(idx_hbm.at[pl.ds(base0, SC_CHUNK)], idx_v0)
            g0 = pltpu.async_copy(table_hbm.at[idx_v0], rows_v0, g_sem0)
            pltpu.sync_copy(idx_hbm.at[pl.ds(base1, SC_CHUNK)], idx_v1)
            g1 = pltpu.async_copy(table_hbm.at[idx_v1], rows_v1, g_sem1)
            g0.wait()
            w0 = pltpu.async_copy(rows_v0, out_hbm.at[pl.ds(base0, SC_CHUNK)],
                                  w_sem0)
            g1.wait()
            w1 = pltpu.async_copy(rows_v1, out_hbm.at[pl.ds(base1, SC_CHUNK)],
                                  w_sem1)
            w0.wait()
            w1.wait()

    return gather_kernel(table, idx)


def _sc_dual_row_gather(table, idx_a, idx_b, chunk=SC_CHUNK):
    d = table.shape[1]
    dtype = table.dtype
    n_rows = idx_a.shape[0]
    rows_per_w = n_rows // NW
    n_chunks = rows_per_w // chunk
    mesh = plsc.VectorSubcoreMesh(core_axis_name="c", subcore_axis_name="s")

    @functools.partial(
        pl.kernel,
        out_type=(jax.ShapeDtypeStruct((n_rows, d), dtype),
                  jax.ShapeDtypeStruct((n_rows, d), dtype)),
        mesh=mesh,
        scratch_types=[
            pltpu.VMEM((chunk,), jnp.int32),
            pltpu.VMEM((chunk,), jnp.int32),
            pltpu.VMEM((chunk, d), dtype),
            pltpu.VMEM((chunk, d), dtype),
            pltpu.SemaphoreType.DMA,
            pltpu.SemaphoreType.DMA,
            pltpu.SemaphoreType.DMA,
            pltpu.SemaphoreType.DMA,
        ],
    )
    def dual_gather_kernel(table_hbm, ia_hbm, ib_hbm, outa_hbm, outb_hbm,
                           ia_v, ib_v, rows_a, rows_b, ga_sem, gb_sem,
                           wa_sem, wb_sem):
        wid = lax.axis_index("s") * NC + lax.axis_index("c")

        @pl.loop(0, n_chunks)
        def _(c):
            base = wid * rows_per_w + c * chunk
            pltpu.sync_copy(ia_hbm.at[pl.ds(base, chunk)], ia_v)
            ga = pltpu.async_copy(table_hbm.at[ia_v], rows_a, ga_sem)
            pltpu.sync_copy(ib_hbm.at[pl.ds(base, chunk)], ib_v)
            gb = pltpu.async_copy(table_hbm.at[ib_v], rows_b, gb_sem)
            ga.wait()
            wa = pltpu.async_copy(rows_a, outa_hbm.at[pl.ds(base, chunk)],
                                  wa_sem)
            gb.wait()
            wb = pltpu.async_copy(rows_b, outb_hbm.at[pl.ds(base, chunk)],
                                  wb_sem)
            wa.wait()
            wb.wait()

    return dual_gather_kernel(table, idx_a, idx_b)


def _pack_body(x_ref, out_ref):
    half = x_ref.shape[1] // 2
    lo = jax.lax.bitcast_convert_type(
        x_ref[:, :half].astype(jnp.bfloat16), jnp.uint16).astype(jnp.uint32)
    hi = jax.lax.bitcast_convert_type(
        x_ref[:, half:].astype(jnp.bfloat16), jnp.uint16).astype(jnp.uint32)
    out_ref[...] = jax.lax.bitcast_convert_type(lo | (hi << 16), jnp.int32)


def _sc_dispatch_scatter(hs, pos_a, pos_b, capacity, chunk=SC_CHUNK):
    n_tok, d = hs.shape
    rows_per_w = n_tok // NW
    n_chunks = rows_per_w // chunk
    mesh = plsc.VectorSubcoreMesh(core_axis_name="c", subcore_axis_name="s")

    @functools.partial(
        pl.kernel,
        out_type=jax.ShapeDtypeStruct((capacity, d), hs.dtype),
        mesh=mesh,
        scratch_types=[
            pltpu.VMEM((chunk,), jnp.int32),
            pltpu.VMEM((chunk,), jnp.int32),
            pltpu.VMEM((chunk,), jnp.int32),
            pltpu.VMEM((chunk,), jnp.int32),
            pltpu.VMEM((chunk, d), hs.dtype),
            pltpu.VMEM((chunk, d), hs.dtype),
            pltpu.SemaphoreType.DMA,
            pltpu.SemaphoreType.DMA,
            pltpu.SemaphoreType.DMA,
            pltpu.SemaphoreType.DMA,
            pltpu.SemaphoreType.DMA,
            pltpu.SemaphoreType.DMA,
        ],
    )
    def scatter_kernel(hs_hbm, pa_hbm, pb_hbm, out_hbm, ia0, ib0, ia1, ib1,
                       rows0, rows1, r_sem0, r_sem1, a_sem0, a_sem1, b_sem0,
                       b_sem1):
        wid = lax.axis_index("s") * NC + lax.axis_index("c")
        idx_a = (ia0, ia1)
        idx_b = (ib0, ib1)
        rows = (rows0, rows1)
        r_sem = (r_sem0, r_sem1)
        a_sem = (a_sem0, a_sem1)
        b_sem = (b_sem0, b_sem1)

        @pl.loop(0, n_chunks, step=2)
        def _(c):
            scats = []
            for k in range(2):
                base = wid * rows_per_w + c * chunk + k * chunk
                r = pltpu.async_copy(hs_hbm.at[pl.ds(base, chunk)],
                                     rows[k], r_sem[k])
                pltpu.sync_copy(pa_hbm.at[pl.ds(base, chunk)], idx_a[k])
                pltpu.sync_copy(pb_hbm.at[pl.ds(base, chunk)], idx_b[k])
                r.wait()
                scats.append(pltpu.async_copy(rows[k], out_hbm.at[idx_a[k]],
                                              a_sem[k]))
                scats.append(pltpu.async_copy(rows[k], out_hbm.at[idx_b[k]],
                                              b_sem[k]))
            for cp in scats:
                cp.wait()

    return scatter_kernel(hs, pos_a, pos_b)


def _grouped_mlp_body(meta_ref, x_ref, wgu_ref, wd_ref, out_ref):
    b = pl.program_id(0)
    e = meta_ref[b]

    @pl.when(e < NUM_EXPERTS)
    def _compute():
        inter = wd_ref.shape[1]
        khalf = wgu_ref.shape[1] // 2
        x_lo, x_hi = _unpack_bf16_pair_raw(x_ref[...])
        gu = jnp.dot(x_lo, wgu_ref[0, :khalf].astype(jnp.bfloat16),
                     preferred_element_type=jnp.float32)
        gu = gu + jnp.dot(x_hi, wgu_ref[0, khalf:].astype(jnp.bfloat16),
                          preferred_element_type=jnp.float32)
        gate = gu[:, :inter]
        up = gu[:, inter:]
        h = (gate * jax.nn.sigmoid(gate)) * up
        out = jnp.dot(h.astype(jnp.bfloat16), wd_ref[0].astype(jnp.bfloat16),
                      preferred_element_type=jnp.float32)
        half = out.shape[1] // 2
        lo = jax.lax.bitcast_convert_type(
            out[:, :half].astype(jnp.bfloat16), jnp.uint16).astype(jnp.uint32)
        hi = jax.lax.bitcast_convert_type(
            out[:, half:].astype(jnp.bfloat16), jnp.uint16).astype(jnp.uint32)
        out_ref[...] = jax.lax.bitcast_convert_type(lo | (hi << 16),
                                                    jnp.int32)


def _unpack_bf16_pair_raw(p):
    u = jax.lax.bitcast_convert_type(p, jnp.uint32)
    lo = jax.lax.bitcast_convert_type((u & 0xFFFF).astype(jnp.uint16),
                                      jnp.bfloat16)
    hi = jax.lax.bitcast_convert_type((u >> 16).astype(jnp.uint16),
                                      jnp.bfloat16)
    return lo, hi


def _unpack_bf16_pair(p):
    lo, hi = _unpack_bf16_pair_raw(p)
    return lo.astype(jnp.float32), hi.astype(jnp.float32)


def _pair_add_body(a_ref, b_ref, w_ref, out_ref):
    w = w_ref[...]
    half = out_ref.shape[1] // 2
    a_lo, a_hi = _unpack_bf16_pair(a_ref[...])
    b_lo, b_hi = _unpack_bf16_pair(b_ref[...])
    out_ref[:, :half] = a_lo * w[:, 0:1] + b_lo * w[:, 1:2]
    out_ref[:, half:] = a_hi * w[:, 0:1] + b_hi * w[:, 1:2]


def kernel(hidden_states, routing_weights, router_indices, gate_up_proj,
           down_proj):
    b, s, hidden = hidden_states.shape
    n_tok = b * s
    num_experts, _, two_inter = gate_up_proj.shape
    inter = two_inter // 2
    n_assign = n_tok * TOP_K
    capacity = n_assign + num_experts * T_BLOCK
    nb = capacity // T_BLOCK

    hs = hidden_states.reshape(n_tok, hidden)
    rw = routing_weights.reshape(n_tok, num_experts)
    ri = router_indices.reshape(n_tok, TOP_K).astype(jnp.int32)

    eids = jnp.arange(num_experts, dtype=jnp.int32)
    sel = ri[:, :, None] == eids[None, None, :]
    w_pair = jnp.sum(jnp.where(sel, rw[:, None, :], 0.0), axis=2)
    dup = ri[:, 1] == ri[:, 0]
    w_pair = jnp.concatenate(
        [w_pair[:, 0:1], jnp.where(dup[:, None], 0.0, w_pair[:, 1:2])], axis=1)

    onehot = sel.reshape(n_assign, num_experts)
    csum = jnp.cumsum(onehot.astype(jnp.int32), axis=0)
    counts = csum[-1]
    rank_a = jnp.sum(jnp.where(onehot, csum, 0), axis=1) - 1
    padded_counts = ((counts + T_BLOCK - 1) // T_BLOCK) * T_BLOCK
    padded_starts = jnp.concatenate(
        [jnp.zeros((1,), jnp.int32),
         jnp.cumsum(padded_counts)[:-1].astype(jnp.int32)])
    start_a = jnp.sum(jnp.where(onehot, padded_starts[None, :], 0), axis=1)
    pos_a = (start_a + rank_a).astype(jnp.int32)
    padded_total = jnp.sum(padded_counts)
    blk_rows = jnp.arange(nb, dtype=jnp.int32) * T_BLOCK
    be = jnp.sum((padded_starts[None, :] <= blk_rows[:, None]).astype(
        jnp.int32), axis=1) - 1
    block_meta = jnp.where(blk_rows < padded_total, be,
                           num_experts).astype(jnp.int32)

    pack_block = 512
    hs_packed = pl.pallas_call(
        _pack_body,
        grid=(n_tok // pack_block,),
        in_specs=[pl.BlockSpec((pack_block, hidden), lambda i: (i, 0))],
        out_specs=pl.BlockSpec((pack_block, hidden // 2), lambda i: (i, 0)),
        out_shape=jax.ShapeDtypeStruct((n_tok, hidden // 2), jnp.int32),
    )(hs)
    pos_2 = pos_a.reshape(n_tok, TOP_K)
    x_sorted = _sc_dispatch_scatter(hs_packed, pos_2[:, 0], pos_2[:, 1],
                                    capacity, chunk=32)

    grid_spec = pltpu.PrefetchScalarGridSpec(
        num_scalar_prefetch=1,
        grid=(nb,),
        in_specs=[
            pl.BlockSpec((T_BLOCK, hidden // 2), lambda i, m: (i, 0)),
            pl.BlockSpec((1, hidden, two_inter),
                         lambda i, m: (jnp.minimum(m[i], NUM_EXPERTS - 1), 0, 0)),
            pl.BlockSpec((1, inter, hidden),
                         lambda i, m: (jnp.minimum(m[i], NUM_EXPERTS - 1), 0, 0)),
        ],
        out_specs=pl.BlockSpec((T_BLOCK, hidden // 2), lambda i, m: (i, 0)),
    )
    out_sorted = pl.pallas_call(
        _grouped_mlp_body,
        grid_spec=grid_spec,
        out_shape=jax.ShapeDtypeStruct((capacity, hidden // 2), jnp.int32),
    )(block_meta, x_sorted, gate_up_proj, down_proj)

    out_a, out_b = _sc_dual_row_gather(out_sorted, pos_2[:, 0], pos_2[:, 1],
                                       chunk=32)

    pair_block = 512
    final = pl.pallas_call(
        _pair_add_body,
        grid=(n_tok // pair_block,),
        in_specs=[
            pl.BlockSpec((pair_block, hidden // 2), lambda i: (i, 0)),
            pl.BlockSpec((pair_block, hidden // 2), lambda i: (i, 0)),
            pl.BlockSpec((pair_block, TOP_K), lambda i: (i, 0)),
        ],
        out_specs=pl.BlockSpec((pair_block, hidden), lambda i: (i, 0)),
        out_shape=jax.ShapeDtypeStruct((n_tok, hidden), jnp.float32),
    )(out_a, out_b, w_pair)

    return final.reshape(b, s, hidden)
